# baseline (speedup 1.0000x reference)
import jax
import jax.numpy as jnp
import numpy as np
from jax.experimental import pallas as pl

B, N, D_IN = 2, 96, 32
MUL = 16
NB, H = 3, 64
D_MID = 3 * MUL
D_OUT = 16
PB = N * N
BN = B * N
SCALE = 1.0 / np.sqrt(float(D_IN) * float(N))


def _sig(x):
    return 1.0 / (1.0 + jnp.exp(-x))


def _swish(x):
    return x * _sig(x)


def _net_kernel(g2, x0t,
                w0_1, b0_1, w1_1, b1_1, w2t_1, bn1g, bn1b,
                w0_2, b0_2, w1_2, b1_2, w2t_2, bn2g, bn2b,
                w0_3, b0_3, w1_3, b1_3, w2t_3,
                out_ref):
    basis_parts = []
    for bb in range(B):
        gb = g2[bb * N:(bb + 1) * N, :]
        gj = jnp.broadcast_to(gb[:, None, :], (N, N, 3)).reshape(PB, 3)
        gi = jnp.broadcast_to(gb[None, :, :], (N, N, 3)).reshape(PB, 3)
        d = gi - gj
        r = jnp.sqrt(jnp.sum(d * d, axis=1, keepdims=True) + 1e-12)
        centers = jax.lax.broadcasted_iota(
            jnp.int32, (1, NB), 1).astype(jnp.float32) * (1.0 / (NB - 1))
        t = (r - centers) * float(NB - 1)
        basis_parts.append(jnp.exp(-(t * t)))
    basis = jnp.concatenate(basis_parts, axis=0)

    def conv(xt, w0, b0, w1, b1, w2t, d_out):
        h0 = _swish(jnp.dot(basis, w0[...],
                            preferred_element_type=jnp.float32) + b0[...])
        h1 = _swish(jnp.dot(h0, w1[...], precision=jax.lax.Precision.HIGHEST,
                            preferred_element_type=jnp.float32) + b1[...])
        yt_parts = []
        for bb in range(B):
            hb = h1[bb * PB:(bb + 1) * PB, :].reshape(N, N, H)
            hf = jnp.swapaxes(hb, 1, 2).reshape(N * H, N)
            m = jnp.dot(w2t[...], xt[:, bb * N:(bb + 1) * N],
                        preferred_element_type=jnp.float32)
            mt = jnp.swapaxes(m.reshape(d_out, H, N), 1, 2).reshape(d_out, N * H)
            yt_parts.append(jnp.dot(mt, hf, precision=jax.lax.Precision.HIGHEST,
                                    preferred_element_type=jnp.float32))
        return jnp.concatenate(yt_parts, axis=1)

    def bnorm(y, g, bta):
        mu = jnp.mean(y, axis=1, keepdims=True)
        dv = y - mu
        var = jnp.mean(dv * dv, axis=1, keepdims=True)
        return dv * jax.lax.rsqrt(var + 1e-5) * g[...] + bta[...]

    def gated(y):
        s = y[:MUL, :]
        gg = y[MUL:2 * MUL, :]
        ns = y[2 * MUL:, :]
        return jnp.concatenate([_swish(s), _sig(gg) * ns], axis=0)

    y = gated(bnorm(conv(x0t, w0_1, b0_1, w1_1, b1_1, w2t_1, D_MID), bn1g, bn1b))
    y = gated(bnorm(conv(y, w0_2, b0_2, w1_2, b1_2, w2t_2, D_MID), bn2g, bn2b))
    out_ref[...] = conv(y, w0_3, b0_3, w1_3, b1_3, w2t_3, D_OUT)


def _prep_w2(w2, d_out):
    return (w2.reshape(H, d_out, D_IN).transpose(1, 0, 2)
            .reshape(d_out * H, D_IN)) * SCALE


def kernel(input, geometry, r1_w0, r1_b0, r1_w1, r1_b1, r1_w2, bn1_g, bn1_b,
           r2_w0, r2_b0, r2_w1, r2_b1, r2_w2, bn2_g, bn2_b,
           r3_w0, r3_b0, r3_w1, r3_b1, r3_w2):
    g2 = geometry.astype(jnp.float32).reshape(BN, 3)
    x0t = input.reshape(BN, D_IN).T

    args = (
        g2, x0t,
        r1_w0, r1_b0.reshape(1, H), r1_w1, r1_b1.reshape(1, H),
        _prep_w2(r1_w2, D_MID), bn1_g.reshape(D_MID, 1), bn1_b.reshape(D_MID, 1),
        r2_w0, r2_b0.reshape(1, H), r2_w1, r2_b1.reshape(1, H),
        _prep_w2(r2_w2, D_MID), bn2_g.reshape(D_MID, 1), bn2_b.reshape(D_MID, 1),
        r3_w0, r3_b0.reshape(1, H), r3_w1, r3_b1.reshape(1, H),
        _prep_w2(r3_w2, D_OUT),
    )
    out = pl.pallas_call(
        _net_kernel,
        out_shape=jax.ShapeDtypeStruct((D_OUT, BN), jnp.float32),
    )(*args)
    return out.T.reshape(B, N, D_OUT)


# speedup vs baseline: 1.5713x; 1.5713x over previous
import jax
import jax.numpy as jnp
import numpy as np
from jax.experimental import pallas as pl

B, N, D_IN = 2, 96, 32
MUL = 16
NB, H = 3, 64
D_MID = 3 * MUL
D_OUT = 16
PB = N * N
BN = B * N
SCALE = 1.0 / np.sqrt(float(D_IN) * float(N))


def _sig(x):
    return 1.0 / (1.0 + jnp.exp(-x))


def _swish(x):
    return x * _sig(x)


def _net_kernel(g2, x0t,
                w0_1, b0_1, w1_1, b1_1, w2t_1, bn1g, bn1b,
                w0_2, b0_2, w1_2, b1_2, w2t_2, bn2g, bn2b,
                w0_3, b0_3, w1_3, b1_3, w2t_3,
                out_ref):
    basis_parts = []
    for bb in range(B):
        gb = g2[bb * N:(bb + 1) * N, :]
        gj = jnp.broadcast_to(gb[:, None, :], (N, N, 3)).reshape(PB, 3)
        gi = jnp.broadcast_to(gb[None, :, :], (N, N, 3)).reshape(PB, 3)
        d = gi - gj
        r = jnp.sqrt(jnp.sum(d * d, axis=1, keepdims=True) + 1e-12)
        centers = jax.lax.broadcasted_iota(
            jnp.int32, (1, NB), 1).astype(jnp.float32) * (1.0 / (NB - 1))
        t = (r - centers) * float(NB - 1)
        basis_parts.append(jnp.exp(-(t * t)))
    basis = jnp.concatenate(basis_parts, axis=0)

    def conv(xt, w0, b0, w1, b1, w2t, d_out):
        h0 = _swish(jnp.dot(basis, w0[...],
                            preferred_element_type=jnp.float32) + b0[...])
        h1 = _swish(jnp.dot(h0, w1[...],
                            preferred_element_type=jnp.float32) + b1[...])
        yt_parts = []
        for bb in range(B):
            hb = h1[bb * PB:(bb + 1) * PB, :].reshape(N, N, H)
            hf = jnp.swapaxes(hb, 1, 2).reshape(N * H, N)
            m = jnp.dot(w2t[...], xt[:, bb * N:(bb + 1) * N],
                        preferred_element_type=jnp.float32)
            mt = jnp.swapaxes(m.reshape(d_out, H, N), 1, 2).reshape(d_out, N * H)
            yt_parts.append(jnp.dot(mt, hf, precision=jax.lax.Precision.HIGHEST,
                                    preferred_element_type=jnp.float32))
        return jnp.concatenate(yt_parts, axis=1)

    def bnorm(y, g, bta):
        mu = jnp.mean(y, axis=1, keepdims=True)
        dv = y - mu
        var = jnp.mean(dv * dv, axis=1, keepdims=True)
        return dv * jax.lax.rsqrt(var + 1e-5) * g[...] + bta[...]

    def gated(y):
        s = y[:MUL, :]
        gg = y[MUL:2 * MUL, :]
        ns = y[2 * MUL:, :]
        return jnp.concatenate([_swish(s), _sig(gg) * ns], axis=0)

    y = gated(bnorm(conv(x0t, w0_1, b0_1, w1_1, b1_1, w2t_1, D_MID), bn1g, bn1b))
    y = gated(bnorm(conv(y, w0_2, b0_2, w1_2, b1_2, w2t_2, D_MID), bn2g, bn2b))
    out_ref[...] = conv(y, w0_3, b0_3, w1_3, b1_3, w2t_3, D_OUT)


def _prep_w2(w2, d_out):
    return (w2.reshape(H, d_out, D_IN).transpose(1, 0, 2)
            .reshape(d_out * H, D_IN)) * SCALE


def kernel(input, geometry, r1_w0, r1_b0, r1_w1, r1_b1, r1_w2, bn1_g, bn1_b,
           r2_w0, r2_b0, r2_w1, r2_b1, r2_w2, bn2_g, bn2_b,
           r3_w0, r3_b0, r3_w1, r3_b1, r3_w2):
    g2 = geometry.astype(jnp.float32).reshape(BN, 3)
    x0t = input.reshape(BN, D_IN).T

    args = (
        g2, x0t,
        r1_w0, r1_b0.reshape(1, H), r1_w1, r1_b1.reshape(1, H),
        _prep_w2(r1_w2, D_MID), bn1_g.reshape(D_MID, 1), bn1_b.reshape(D_MID, 1),
        r2_w0, r2_b0.reshape(1, H), r2_w1, r2_b1.reshape(1, H),
        _prep_w2(r2_w2, D_MID), bn2_g.reshape(D_MID, 1), bn2_b.reshape(D_MID, 1),
        r3_w0, r3_b0.reshape(1, H), r3_w1, r3_b1.reshape(1, H),
        _prep_w2(r3_w2, D_OUT),
    )
    out = pl.pallas_call(
        _net_kernel,
        out_shape=jax.ShapeDtypeStruct((D_OUT, BN), jnp.float32),
    )(*args)
    return out.T.reshape(B, N, D_OUT)


# speedup vs baseline: 2.0664x; 1.3151x over previous
import jax
import jax.numpy as jnp
import numpy as np
from jax.experimental import pallas as pl

B, N, D_IN = 2, 96, 32
MUL = 16
NB, H = 3, 64
D_MID = 3 * MUL
D_OUT = 16
PB = N * N
BN = B * N
SCALE = 1.0 / np.sqrt(float(D_IN) * float(N))


def _sig(x):
    return 1.0 / (1.0 + jnp.exp(-x))


def _swish(x):
    return x * _sig(x)


def _net_kernel(g2, x0t,
                w0_1, b0_1, w1_1, b1_1, w2t_1, bn1g, bn1b,
                w0_2, b0_2, w1_2, b1_2, w2t_2, bn2g, bn2b,
                w0_3, b0_3, w1_3, b1_3, w2t_3,
                out_ref):
    basis_parts = []
    for bb in range(B):
        gb = g2[bb * N:(bb + 1) * N, :]
        gj = jnp.broadcast_to(gb[:, None, :], (N, N, 3)).reshape(PB, 3)
        gi = jnp.broadcast_to(gb[None, :, :], (N, N, 3)).reshape(PB, 3)
        d = gi - gj
        r = jnp.sqrt(jnp.sum(d * d, axis=1, keepdims=True) + 1e-12)
        centers = jax.lax.broadcasted_iota(
            jnp.int32, (1, NB), 1).astype(jnp.float32) * (1.0 / (NB - 1))
        t = (r - centers) * float(NB - 1)
        basis_parts.append(jnp.exp(-(t * t)))
    basis = jnp.concatenate(basis_parts, axis=0)

    def conv(xt, w0, b0, w1, b1, w2t, d_out):
        h0 = _swish(jnp.dot(basis, w0[...],
                            preferred_element_type=jnp.float32) + b0[...])
        h1 = _swish(jnp.dot(h0, w1[...],
                            preferred_element_type=jnp.float32) + b1[...])
        yt_parts = []
        for bb in range(B):
            hb = h1[bb * PB:(bb + 1) * PB, :].reshape(N, N, H)
            hf = jnp.swapaxes(hb, 1, 2).reshape(N * H, N)
            m = jnp.dot(w2t[...], xt[:, bb * N:(bb + 1) * N],
                        preferred_element_type=jnp.float32)
            mt = jnp.swapaxes(m.reshape(d_out, H, N), 1, 2).reshape(d_out, N * H)
            yt_parts.append(jnp.dot(mt, hf,
                                    preferred_element_type=jnp.float32))
        return jnp.concatenate(yt_parts, axis=1)

    def bnorm(y, g, bta):
        mu = jnp.mean(y, axis=1, keepdims=True)
        dv = y - mu
        var = jnp.mean(dv * dv, axis=1, keepdims=True)
        return dv * jax.lax.rsqrt(var + 1e-5) * g[...] + bta[...]

    def gated(y):
        s = y[:MUL, :]
        gg = y[MUL:2 * MUL, :]
        ns = y[2 * MUL:, :]
        return jnp.concatenate([_swish(s), _sig(gg) * ns], axis=0)

    y = gated(bnorm(conv(x0t, w0_1, b0_1, w1_1, b1_1, w2t_1, D_MID), bn1g, bn1b))
    y = gated(bnorm(conv(y, w0_2, b0_2, w1_2, b1_2, w2t_2, D_MID), bn2g, bn2b))
    out_ref[...] = conv(y, w0_3, b0_3, w1_3, b1_3, w2t_3, D_OUT)


def _prep_w2(w2, d_out):
    return (w2.reshape(H, d_out, D_IN).transpose(1, 0, 2)
            .reshape(d_out * H, D_IN)) * SCALE


def kernel(input, geometry, r1_w0, r1_b0, r1_w1, r1_b1, r1_w2, bn1_g, bn1_b,
           r2_w0, r2_b0, r2_w1, r2_b1, r2_w2, bn2_g, bn2_b,
           r3_w0, r3_b0, r3_w1, r3_b1, r3_w2):
    g2 = geometry.astype(jnp.float32).reshape(BN, 3)
    x0t = input.reshape(BN, D_IN).T

    args = (
        g2, x0t,
        r1_w0, r1_b0.reshape(1, H), r1_w1, r1_b1.reshape(1, H),
        _prep_w2(r1_w2, D_MID), bn1_g.reshape(D_MID, 1), bn1_b.reshape(D_MID, 1),
        r2_w0, r2_b0.reshape(1, H), r2_w1, r2_b1.reshape(1, H),
        _prep_w2(r2_w2, D_MID), bn2_g.reshape(D_MID, 1), bn2_b.reshape(D_MID, 1),
        r3_w0, r3_b0.reshape(1, H), r3_w1, r3_b1.reshape(1, H),
        _prep_w2(r3_w2, D_OUT),
    )
    out = pl.pallas_call(
        _net_kernel,
        out_shape=jax.ShapeDtypeStruct((D_OUT, BN), jnp.float32),
    )(*args)
    return out.T.reshape(B, N, D_OUT)
